# Initial kernel scaffold; baseline (speedup 1.0000x reference)
import jax
import jax.numpy as jnp
from jax.experimental import pallas as pl


def kernel(ps, pt, bool_masked_pos):
    raise NotImplementedError("write your pallas kernel here")


# speedup vs baseline: 1.0030x; 1.0030x over previous
import jax
import jax.numpy as jnp
from jax.experimental import pallas as pl
from jax.experimental.pallas import tpu as pltpu

_B, _N, _D = 32, 256, 4096
_T = _B * _N
_BLK = 256


def _ce_body(ps_ref, pt_ref, m_ref, out_ref, s_acc, c_acc):
    i = pl.program_id(0)

    @pl.when(i == 0)
    def _init():
        s_acc[0] = 0.0
        c_acc[0] = 0.0

    ps = ps_ref[...]
    pt = pt_ref[...]
    m = m_ref[...]
    per_tok = -(pt * jnp.log(ps)).sum(axis=-1)
    s_acc[0] += (per_tok * m).sum()
    c_acc[0] += m.sum()

    @pl.when(i == pl.num_programs(0) - 1)
    def _fin():
        out_ref[0, 0] = s_acc[0] / jnp.maximum(c_acc[0], 1.0)


def kernel(ps, pt, bool_masked_pos):
    ps2 = ps.reshape(_T, _D)
    pt2 = pt.reshape(_T, _D)
    m = bool_masked_pos.reshape(_T).astype(jnp.float32)
    out = pl.pallas_call(
        _ce_body,
        grid=(_T // _BLK,),
        in_specs=[
            pl.BlockSpec((_BLK, _D), lambda i: (i, 0)),
            pl.BlockSpec((_BLK, _D), lambda i: (i, 0)),
            pl.BlockSpec((_BLK,), lambda i: (i,)),
        ],
        out_specs=pl.BlockSpec(memory_space=pltpu.SMEM),
        out_shape=jax.ShapeDtypeStruct((1, 1), jnp.float32),
        scratch_shapes=[
            pltpu.SMEM((1,), jnp.float32),
            pltpu.SMEM((1,), jnp.float32),
        ],
    )(ps2, pt2, m)
    return out[0, 0]
